# baseline (device time: 19036 ns/iter reference)
import jax
import jax.numpy as jnp
from jax import lax
from jax.experimental import pallas as pl
from jax.experimental.pallas import tpu as pltpu

NC = 4


def kernel(x, dy):
    m, d = x.shape
    _, f = dy.shape
    d_half = d // 2
    f_half = f // 2
    cw = f_half // NC

    def body(x_hbm, dy_hbm, out_hbm, x_vmem, dyf_vmem, dy_bf_ref,
             p1s_ref, p1r_ref, res_ref, in_sems, s1, r1, s2, r2, so):
        my_x = lax.axis_index("x")
        my_y = lax.axis_index("y")
        other_x = 1 - my_x
        other_y = 1 - my_y

        cp_x = pltpu.make_async_copy(x_hbm, x_vmem, in_sems.at[0])
        cp_x.start()
        cp_dy = pltpu.make_async_copy(
            dy_hbm.at[:, pl.ds(my_x * f_half, f_half)], dyf_vmem,
            in_sems.at[1],
        )
        cp_dy.start()

        barrier = pltpu.get_barrier_semaphore()
        pl.semaphore_signal(barrier, inc=1, device_id=(my_x, other_y),
                            device_id_type=pl.DeviceIdType.MESH)
        pl.semaphore_signal(barrier, inc=1, device_id=(other_x, my_y),
                            device_id_type=pl.DeviceIdType.MESH)
        pl.semaphore_wait(barrier, 2)

        cp_x.wait()
        x_send = x_vmem[:, pl.ds(other_y * d_half, d_half)].astype(
            jnp.bfloat16
        )
        x_keep = x_vmem[:, pl.ds(my_y * d_half, d_half)].astype(jnp.bfloat16)
        cp_dy.wait()
        dy_bf_ref[...] = dyf_vmem[...].astype(jnp.bfloat16)

        dims = (((0,), (0,)), ((), ()))

        rdma1 = []
        for c in range(NC):
            p1s_ref[c] = lax.dot_general(
                x_send, dy_bf_ref[:, c * cw:(c + 1) * cw], dims,
                preferred_element_type=jnp.float32,
            ).astype(jnp.bfloat16)
            r = pltpu.make_async_remote_copy(
                src_ref=p1s_ref.at[c], dst_ref=p1r_ref.at[c],
                send_sem=s1.at[c], recv_sem=r1.at[c],
                device_id=(my_x, other_y),
                device_id_type=pl.DeviceIdType.MESH,
            )
            r.start()
            rdma1.append(r)

        rdma2 = []
        outcp = []
        for c in range(NC):
            pk = lax.dot_general(
                x_keep, dy_bf_ref[:, c * cw:(c + 1) * cw], dims,
                preferred_element_type=jnp.float32,
            )
            rdma1[c].wait_recv()
            res_ref[:, c * cw:(c + 1) * cw] = (
                pk + p1r_ref[c].astype(jnp.float32)
            ).astype(jnp.bfloat16)
            r = pltpu.make_async_remote_copy(
                src_ref=res_ref.at[:, pl.ds(c * cw, cw)],
                dst_ref=out_hbm.at[:, pl.ds(my_x * f_half + c * cw, cw)],
                send_sem=s2.at[c], recv_sem=r2.at[c],
                device_id=(other_x, my_y),
                device_id_type=pl.DeviceIdType.MESH,
            )
            r.start()
            rdma2.append(r)
            cp = pltpu.make_async_copy(
                res_ref.at[:, pl.ds(c * cw, cw)],
                out_hbm.at[:, pl.ds(my_x * f_half + c * cw, cw)],
                so.at[c],
            )
            cp.start()
            outcp.append(cp)

        for c in range(NC):
            outcp[c].wait()
            rdma2[c].wait_recv()
        for c in range(NC):
            rdma1[c].wait_send()
            rdma2[c].wait_send()

    return pl.pallas_call(
        body,
        out_shape=jax.ShapeDtypeStruct((d_half, f), jnp.bfloat16),
        in_specs=[
            pl.BlockSpec(memory_space=pl.ANY),
            pl.BlockSpec(memory_space=pl.ANY),
        ],
        out_specs=pl.BlockSpec(memory_space=pl.ANY),
        scratch_shapes=[
            pltpu.VMEM((m, d), jnp.float32),
            pltpu.VMEM((m, f_half), jnp.float32),
            pltpu.VMEM((m, f_half), jnp.bfloat16),
            pltpu.VMEM((NC, d_half, cw), jnp.bfloat16),
            pltpu.VMEM((NC, d_half, cw), jnp.bfloat16),
            pltpu.VMEM((d_half, f_half), jnp.bfloat16),
            pltpu.SemaphoreType.DMA((2,)),
            pltpu.SemaphoreType.DMA((NC,)),
            pltpu.SemaphoreType.DMA((NC,)),
            pltpu.SemaphoreType.DMA((NC,)),
            pltpu.SemaphoreType.DMA((NC,)),
            pltpu.SemaphoreType.DMA((NC,)),
        ],
        compiler_params=pltpu.CompilerParams(collective_id=0),
    )(x, dy)


# device time: 19002 ns/iter; 1.0018x vs baseline; 1.0018x over previous
import jax
import jax.numpy as jnp
from jax import lax
from jax.experimental import pallas as pl
from jax.experimental.pallas import tpu as pltpu

NC = 4


def kernel(x, dy):
    m, d = x.shape
    _, f = dy.shape
    d_half = d // 2
    f_half = f // 2
    cw = f_half // NC

    def body(x_hbm, dy_hbm, out_hbm, x_vmem, dyf_vmem, dy_bf_ref,
             p1s_ref, p1r_ref, res_ref, in_sems, s1, r1, s2, r2, so):
        my_x = lax.axis_index("x")
        my_y = lax.axis_index("y")
        other_x = 1 - my_x
        other_y = 1 - my_y

        cp_x = pltpu.make_async_copy(x_hbm, x_vmem, in_sems.at[0])
        cp_x.start()
        cp_dy = pltpu.make_async_copy(
            dy_hbm.at[:, pl.ds(my_x * f_half, f_half)], dyf_vmem,
            in_sems.at[1],
        )
        cp_dy.start()

        barrier = pltpu.get_barrier_semaphore()
        pl.semaphore_signal(barrier, inc=1, device_id=(my_x, other_y),
                            device_id_type=pl.DeviceIdType.MESH)
        pl.semaphore_signal(barrier, inc=1, device_id=(other_x, my_y),
                            device_id_type=pl.DeviceIdType.MESH)
        pl.semaphore_wait(barrier, 2)

        cp_x.wait()
        x_send = x_vmem[:, pl.ds(other_y * d_half, d_half)].astype(
            jnp.bfloat16
        )
        x_keep = x_vmem[:, pl.ds(my_y * d_half, d_half)].astype(jnp.bfloat16)
        cp_dy.wait()
        dy_bf_ref[...] = dyf_vmem[...].astype(jnp.bfloat16)

        dims = (((0,), (0,)), ((), ()))

        rdma1 = []
        for c in range(NC):
            p1s_ref[c] = lax.dot_general(
                x_send, dy_bf_ref[:, c * cw:(c + 1) * cw], dims,
                preferred_element_type=jnp.float32,
            ).astype(jnp.bfloat16)
            r = pltpu.make_async_remote_copy(
                src_ref=p1s_ref.at[c], dst_ref=p1r_ref.at[c],
                send_sem=s1.at[c], recv_sem=r1.at[c],
                device_id=(my_x, other_y),
                device_id_type=pl.DeviceIdType.MESH,
            )
            r.start()
            rdma1.append(r)

        rdma2 = []
        outcp = []
        for c in range(NC):
            pk = lax.dot_general(
                x_keep, dy_bf_ref[:, c * cw:(c + 1) * cw], dims,
                preferred_element_type=jnp.float32,
            )
            rdma1[c].wait_recv()
            res_ref[:, c * cw:(c + 1) * cw] = (
                pk + p1r_ref[c].astype(jnp.float32)
            ).astype(jnp.bfloat16)
            r = pltpu.make_async_remote_copy(
                src_ref=res_ref.at[:, pl.ds(c * cw, cw)],
                dst_ref=out_hbm.at[:, pl.ds(my_x * f_half + c * cw, cw)],
                send_sem=s2.at[c], recv_sem=r2.at[c],
                device_id=(other_x, my_y),
                device_id_type=pl.DeviceIdType.MESH,
            )
            r.start()
            rdma2.append(r)
            cp = pltpu.make_async_copy(
                res_ref.at[:, pl.ds(c * cw, cw)],
                out_hbm.at[:, pl.ds(my_x * f_half + c * cw, cw)],
                so.at[c],
            )
            cp.start()
            outcp.append(cp)

        for c in range(NC):
            outcp[c].wait()
            rdma2[c].wait_recv()
        for c in range(NC):
            rdma1[c].wait_send()
            rdma2[c].wait_send()

    return pl.pallas_call(
        body,
        out_shape=jax.ShapeDtypeStruct((d_half, f), jnp.bfloat16),
        in_specs=[
            pl.BlockSpec(memory_space=pltpu.MemorySpace.HBM),
            pl.BlockSpec(memory_space=pltpu.MemorySpace.HBM),
        ],
        out_specs=pl.BlockSpec(memory_space=pltpu.MemorySpace.HBM),
        scratch_shapes=[
            pltpu.VMEM((m, d), jnp.float32),
            pltpu.VMEM((m, f_half), jnp.float32),
            pltpu.VMEM((m, f_half), jnp.bfloat16),
            pltpu.VMEM((NC, d_half, cw), jnp.bfloat16),
            pltpu.VMEM((NC, d_half, cw), jnp.bfloat16),
            pltpu.VMEM((d_half, f_half), jnp.bfloat16),
            pltpu.SemaphoreType.DMA((2,)),
            pltpu.SemaphoreType.DMA((NC,)),
            pltpu.SemaphoreType.DMA((NC,)),
            pltpu.SemaphoreType.DMA((NC,)),
            pltpu.SemaphoreType.DMA((NC,)),
            pltpu.SemaphoreType.DMA((NC,)),
        ],
        compiler_params=pltpu.CompilerParams(collective_id=0),
    )(x, dy)
